# Initial kernel scaffold; baseline (speedup 1.0000x reference)
import jax
import jax.numpy as jnp
from jax.experimental import pallas as pl


def kernel(x, token_table, pos_table):
    raise NotImplementedError("write your pallas kernel here")


# speedup vs baseline: 1.4043x; 1.4043x over previous
import functools

import jax
import jax.numpy as jnp
from jax import lax
from jax.experimental import pallas as pl
from jax.experimental.pallas import tpu as pltpu
from jax.experimental.pallas import tpu_sc as plsc

SUB = 100
CH = 1600


def kernel(x, token_table, pos_table):
    B, L = x.shape
    V, D = token_table.shape
    N = B * L
    info = plsc.get_sparse_core_info()
    NC, NS = info.num_cores, info.num_subcores
    NW = NC * NS
    per_w = N // NW
    n_ch = per_w // CH
    n_sub = CH // SUB
    xf = x.reshape(N // SUB, SUB).astype(jnp.int32)

    mesh = plsc.VectorSubcoreMesh(core_axis_name="c", subcore_axis_name="s")

    @functools.partial(
        pl.kernel,
        out_type=jax.ShapeDtypeStruct((N, D), jnp.float32),
        mesh=mesh,
        compiler_params=pltpu.CompilerParams(use_tc_tiling_on_sc=False),
        scratch_types=[
            pltpu.VMEM((n_sub, SUB), jnp.int32),
            pltpu.VMEM((CH, D), jnp.float32),
            pltpu.VMEM((L, D), jnp.float32),
            pltpu.SemaphoreType.DMA,
        ],
    )
    def sc_kernel(x_hbm, tok_hbm, pos_hbm, out_hbm, idx_v, rows_v, pos_v, sem):
        wid = lax.axis_index("s") * NC + lax.axis_index("c")
        base = wid * per_w
        pltpu.sync_copy(pos_hbm, pos_v)

        @pl.loop(0, n_ch)
        def _chunk(g):
            start = pl.multiple_of(base + g * CH, CH)
            pltpu.sync_copy(
                x_hbm.at[pl.ds(pl.multiple_of(start // SUB, n_sub), n_sub)],
                idx_v,
            )
            copies = [
                pltpu.async_copy(
                    tok_hbm.at[idx_v.at[j]],
                    rows_v.at[pl.ds(j * SUB, SUB)],
                    sem,
                )
                for j in range(n_sub)
            ]
            for c in copies:
                c.wait()

            @pl.loop(0, L)
            def _row(r):
                for gg in range(CH // L):
                    for h in range(D // 16):
                        sl = pl.ds(h * 16, 16)
                        rows_v[gg * L + r, sl] = (
                            rows_v[gg * L + r, sl] + pos_v[r, sl]
                        )

            pltpu.sync_copy(rows_v, out_hbm.at[pl.ds(start, CH)])

    out = sc_kernel(xf, token_table, pos_table)
    return out.reshape(B, L, D)


# speedup vs baseline: 1.6405x; 1.1683x over previous
import functools

import jax
import jax.numpy as jnp
from jax import lax
from jax.experimental import pallas as pl
from jax.experimental.pallas import tpu as pltpu
from jax.experimental.pallas import tpu_sc as plsc

KL = 8
CB = 8192


def _table_rowmajor(token_table):
    V, D = token_table.shape
    tokT = token_table.T
    nfull = V // CB
    tail = V - nfull * CB
    tail_rows = tail * D // 128
    tokT_tail = lax.slice(tokT, (0, nfull * CB), (D, V))
    rows_out = (nfull + 1) * (CB * D // 128)

    def merge(xb):
        n = xb.shape[1]
        t = jnp.transpose(xb, (1, 0))
        t3 = t.reshape(n // 4, 4, D)
        parts = [
            lax.squeeze(lax.slice(t3, (0, k, 0), (n // 4, k + 1, D)), [1])
            for k in range(4)
        ]
        return jnp.concatenate(parts, axis=1)

    def body(x_ref, tail_ref, o_ref):
        g = pl.program_id(0)

        @pl.when(g < nfull)
        def _():
            o_ref[...] = merge(x_ref[...])

        @pl.when(g == nfull)
        def _():
            o_ref[pl.ds(0, tail_rows), :] = merge(tail_ref[...])

    return pl.pallas_call(
        body,
        grid=(nfull + 1,),
        in_specs=[
            pl.BlockSpec((D, CB), lambda g: (0, jnp.minimum(g, nfull - 1))),
            pl.BlockSpec((D, tail), lambda g: (0, 0)),
        ],
        out_specs=pl.BlockSpec((CB * D // 128, 128), lambda g: (g, 0)),
        out_shape=jax.ShapeDtypeStruct((rows_out, 128), jnp.float32),
    )(tokT, tokT_tail)


def kernel(x, token_table, pos_table):
    B, L = x.shape
    V, D = token_table.shape
    info = plsc.get_sparse_core_info()
    NC, NS = info.num_cores, info.num_subcores
    NW = NC * NS
    BW = B // NW
    n_it = L // KL
    xT = x.T.astype(jnp.int32)

    tok128 = _table_rowmajor(token_table)
    Vp = tok128.shape[0] * 128 // D
    tok_rm = tok128.reshape(Vp, D)

    mesh = plsc.VectorSubcoreMesh(core_axis_name="c", subcore_axis_name="s")

    @functools.partial(
        pl.kernel,
        out_type=jax.ShapeDtypeStruct((L, B, D), jnp.float32),
        mesh=mesh,
        compiler_params=pltpu.CompilerParams(use_tc_tiling_on_sc=False),
        scratch_types=[
            pltpu.VMEM((KL, BW), jnp.int32),
            pltpu.VMEM((KL, BW, D), jnp.float32),
            pltpu.VMEM((L, D), jnp.float32),
            pltpu.SemaphoreType.DMA,
        ],
    )
    def sc_kernel(xT_hbm, tok_hbm, pos_hbm, out_hbm, idx_v, rows_v, pos_v, sem):
        wid = lax.axis_index("s") * NC + lax.axis_index("c")
        col0 = pl.multiple_of(wid * BW, BW)
        pltpu.sync_copy(pos_hbm, pos_v)

        @pl.loop(0, n_it)
        def _chunk(it):
            l0 = pl.multiple_of(it * KL, KL)
            pltpu.sync_copy(xT_hbm.at[pl.ds(l0, KL), pl.ds(col0, BW)], idx_v)
            copies = [
                pltpu.async_copy(tok_hbm.at[idx_v.at[j]], rows_v.at[j], sem)
                for j in range(KL)
            ]
            for c in copies:
                c.wait()

            for j in range(KL):
                p0 = pos_v[l0 + j, pl.ds(0, 16)]
                p1 = pos_v[l0 + j, pl.ds(16, 16)]

                @pl.loop(0, BW // 8)
                def _rows(tb, j=j, p0=p0, p1=p1):
                    for tt in range(8):
                        r = tb * 8 + tt
                        rows_v[j, r, pl.ds(0, 16)] = (
                            rows_v[j, r, pl.ds(0, 16)] + p0
                        )
                        rows_v[j, r, pl.ds(16, 16)] = (
                            rows_v[j, r, pl.ds(16, 16)] + p1
                        )

            pltpu.sync_copy(
                rows_v, out_hbm.at[pl.ds(l0, KL), pl.ds(col0, BW)]
            )

    out = sc_kernel(xT, tok_rm, pos_table)
    return out.transpose(1, 0, 2)
